# baseline (speedup 1.0000x reference)
import functools

import jax
import jax.numpy as jnp
from jax.experimental import pallas as pl


def _lsh_kernel(x_ref, r_ref, o_ref, *, rounds, nb_half, length, tl, d_k):
    x = x_ref[0]
    r = r_ref[0]
    nb = rounds * nb_half

    ones = jnp.ones((d_k, 128), jnp.float32)
    n2 = jax.lax.dot_general(x * x, ones, (((1,), (0,)), ((), ())),
                             precision=jax.lax.Precision.HIGHEST,
                             preferred_element_type=jnp.float32)
    nn = jnp.maximum(jnp.sqrt(n2), 1e-12)
    xn = x / nn[:, :d_k]

    mt = jax.lax.dot_general(r, xn, (((0,), (1,)), ((), ())),
                             preferred_element_type=jnp.float32)

    targets = []
    offs = []
    for rd in range(rounds):
        v = mt[rd * nb_half:(rd + 1) * nb_half]
        mp = jnp.max(v, axis=0, keepdims=True)
        mn = jnp.min(v, axis=0, keepdims=True)
        use_pos = mp >= -mn
        targets.append(jnp.broadcast_to(jnp.where(use_pos, mp, mn),
                                        (nb_half, tl)))
        offs.append(jnp.where(use_pos, 0.0, float(nb_half)))
    target_full = jnp.concatenate(targets, axis=0)
    off_full = jnp.concatenate(offs, axis=0)

    row_full = jax.lax.broadcasted_iota(jnp.int32, (nb, tl), 0).astype(jnp.float32)
    contrib = jnp.where(mt == target_full, row_full, 0.0)
    ri = jax.lax.broadcasted_iota(jnp.int32, (rounds, nb), 0)
    ci = jax.lax.broadcasted_iota(jnp.int32, (rounds, nb), 1)
    blk = (ci // nb_half == ri).astype(jnp.float32)
    idxf = jax.lax.dot_general(blk, contrib,
                               (((1,), (0,)), ((), ())),
                               preferred_element_type=jnp.float32)
    base = jax.lax.broadcasted_iota(jnp.int32, (rounds, 1), 0).astype(jnp.float32) * nb_half
    posf = (pl.program_id(1) * tl + jax.lax.broadcasted_iota(
        jnp.int32, (1, tl), 1)).astype(jnp.float32)
    hashf = (idxf - base + off_full) * float(length) + posf
    o_ref[0] = hashf.astype(jnp.int32)


def kernel(inp, rand_matrix, n_buckets):
    batch, length, d_k = inp.shape
    rounds = rand_matrix.shape[2]
    nb_half = rand_matrix.shape[3]
    r2 = rand_matrix.reshape(batch, d_k, rounds * nb_half)

    tl = 4096
    grid = (batch, length // tl)
    out = pl.pallas_call(
        functools.partial(_lsh_kernel, rounds=rounds, nb_half=nb_half,
                          length=length, tl=tl, d_k=d_k),
        grid=grid,
        in_specs=[
            pl.BlockSpec((1, tl, d_k), lambda b, l: (b, l, 0)),
            pl.BlockSpec((1, d_k, rounds * nb_half), lambda b, l: (b, 0, 0)),
        ],
        out_specs=pl.BlockSpec((1, rounds, tl), lambda b, l: (b, 0, l)),
        out_shape=jax.ShapeDtypeStruct((batch, rounds, length), jnp.int32),
    )(inp, r2)
    return out.transpose(0, 2, 1)


# speedup vs baseline: 1.5635x; 1.5635x over previous
import functools

import jax
import jax.numpy as jnp
from jax.experimental import pallas as pl


def _lsh_kernel(x_ref, r_ref, o_ref, *, rounds, nb_half, length, tl, d_k):
    x = x_ref[0]
    r = r_ref[0]
    nb = rounds * nb_half

    norm = jnp.sqrt(jnp.sum(x * x, axis=-1, keepdims=True))
    xn = x / jnp.maximum(norm, 1e-12)

    mt = jax.lax.dot_general(r, xn, (((0,), (1,)), ((), ())),
                             preferred_element_type=jnp.float32)

    targets = []
    offs = []
    for rd in range(rounds):
        v = mt[rd * nb_half:(rd + 1) * nb_half]
        mp = jnp.max(v, axis=0, keepdims=True)
        mn = jnp.min(v, axis=0, keepdims=True)
        use_pos = mp >= -mn
        targets.append(jnp.broadcast_to(jnp.where(use_pos, mp, mn),
                                        (nb_half, tl)))
        offs.append(jnp.where(use_pos, 0.0, float(nb_half)))
    target_full = jnp.concatenate(targets, axis=0)
    off_full = jnp.concatenate(offs, axis=0)

    row_full = jax.lax.broadcasted_iota(jnp.int32, (nb, tl), 0).astype(jnp.float32)
    contrib = jnp.where(mt == target_full, row_full, 0.0)
    ri = jax.lax.broadcasted_iota(jnp.int32, (rounds, nb), 0)
    ci = jax.lax.broadcasted_iota(jnp.int32, (rounds, nb), 1)
    blk = (ci // nb_half == ri).astype(jnp.float32)
    idxf = jax.lax.dot_general(blk, contrib,
                               (((1,), (0,)), ((), ())),
                               preferred_element_type=jnp.float32)
    base = jax.lax.broadcasted_iota(jnp.int32, (rounds, 1), 0).astype(jnp.float32) * nb_half
    posf = (pl.program_id(1) * tl + jax.lax.broadcasted_iota(
        jnp.int32, (1, tl), 1)).astype(jnp.float32)
    hashf = (idxf - base + off_full) * float(length) + posf
    o_ref[0] = hashf.astype(jnp.int32)


def kernel(inp, rand_matrix, n_buckets):
    batch, length, d_k = inp.shape
    rounds = rand_matrix.shape[2]
    nb_half = rand_matrix.shape[3]
    r2 = rand_matrix.reshape(batch, d_k, rounds * nb_half)

    tl = 4096
    grid = (batch, length // tl)
    out = pl.pallas_call(
        functools.partial(_lsh_kernel, rounds=rounds, nb_half=nb_half,
                          length=length, tl=tl, d_k=d_k),
        grid=grid,
        in_specs=[
            pl.BlockSpec((1, tl, d_k), lambda b, l: (b, l, 0)),
            pl.BlockSpec((1, d_k, rounds * nb_half), lambda b, l: (b, 0, 0)),
        ],
        out_specs=pl.BlockSpec((1, rounds, tl), lambda b, l: (b, 0, l)),
        out_shape=jax.ShapeDtypeStruct((batch, rounds, length), jnp.int32),
    )(inp, r2)
    return out.transpose(0, 2, 1)


# speedup vs baseline: 1.6264x; 1.0402x over previous
import functools

import jax
import jax.numpy as jnp
from jax.experimental import pallas as pl


def _lsh_kernel(x_ref, r_ref, o_ref, *, rounds, nb_half, length, tl, d_k, bb):
    nb = rounds * nb_half
    for ib in range(bb):
        _one_batch(x_ref[ib], r_ref[ib], o_ref, ib, rounds=rounds,
                   nb_half=nb_half, length=length, tl=tl, d_k=d_k, nb=nb)


def _one_batch(x, r, o_ref, ib, *, rounds, nb_half, length, tl, d_k, nb):

    norm = jnp.sqrt(jnp.sum(x * x, axis=-1, keepdims=True))
    xn = x / jnp.maximum(norm, 1e-12)

    mt = jax.lax.dot_general(r, xn, (((0,), (1,)), ((), ())),
                             preferred_element_type=jnp.float32)

    targets = []
    offs = []
    for rd in range(rounds):
        v = mt[rd * nb_half:(rd + 1) * nb_half]
        mp = jnp.max(v, axis=0, keepdims=True)
        mn = jnp.min(v, axis=0, keepdims=True)
        use_pos = mp >= -mn
        targets.append(jnp.broadcast_to(jnp.where(use_pos, mp, mn),
                                        (nb_half, tl)))
        offs.append(jnp.where(use_pos, 0.0, float(nb_half)))
    target_full = jnp.concatenate(targets, axis=0)
    off_full = jnp.concatenate(offs, axis=0)

    row_full = jax.lax.broadcasted_iota(jnp.int32, (nb, tl), 0).astype(jnp.float32)
    contrib = jnp.where(mt == target_full, row_full, 0.0)
    ri = jax.lax.broadcasted_iota(jnp.int32, (rounds, nb), 0)
    ci = jax.lax.broadcasted_iota(jnp.int32, (rounds, nb), 1)
    blk = (ci // nb_half == ri).astype(jnp.float32)
    idxf = jax.lax.dot_general(blk, contrib,
                               (((1,), (0,)), ((), ())),
                               preferred_element_type=jnp.float32)
    base = jax.lax.broadcasted_iota(jnp.int32, (rounds, 1), 0).astype(jnp.float32) * nb_half
    posf = jax.lax.broadcasted_iota(
        jnp.int32, (1, tl), 1).astype(jnp.float32)
    hashf = (idxf - base + off_full) * float(length) + posf
    o_ref[ib] = hashf.astype(jnp.int32)


def kernel(inp, rand_matrix, n_buckets):
    batch, length, d_k = inp.shape
    rounds = rand_matrix.shape[2]
    nb_half = rand_matrix.shape[3]
    r2 = rand_matrix.reshape(batch, d_k, rounds * nb_half)

    tl = 4096
    bb = 4
    grid = (batch // bb,)
    out = pl.pallas_call(
        functools.partial(_lsh_kernel, rounds=rounds, nb_half=nb_half,
                          length=length, tl=tl, d_k=d_k, bb=bb),
        grid=grid,
        in_specs=[
            pl.BlockSpec((bb, tl, d_k), lambda b: (b, 0, 0)),
            pl.BlockSpec((bb, d_k, rounds * nb_half), lambda b: (b, 0, 0)),
        ],
        out_specs=pl.BlockSpec((bb, rounds, tl), lambda b: (b, 0, 0)),
        out_shape=jax.ShapeDtypeStruct((batch, rounds, length), jnp.int32),
    )(inp, r2)
    return out.transpose(0, 2, 1)
